# Initial kernel scaffold; baseline (speedup 1.0000x reference)
import jax
import jax.numpy as jnp
from jax.experimental import pallas as pl


def kernel(x_seq, edge_index, edge_attr, W1, b1, W2, b2, W_ih, W_hh, b_ih, b_hh, Wc, bc):
    raise NotImplementedError("write your pallas kernel here")


# speedup vs baseline: 2.6844x; 2.6844x over previous
import functools

import jax
import jax.numpy as jnp
from jax import lax
from jax.experimental import pallas as pl
from jax.experimental.pallas import tpu as pltpu
from jax.experimental.pallas import tpu_sc as plsc

T, N, E, D = 8, 10000, 320000, 128
NC, NS, LANES = 2, 16, 16
NW = NC * NS
NPAD = 10240
RPT = NPAD // NS
EPW = E // NW
K = 80
CHUNKS = EPW // K
ZROWS = 160
RB = 1000
NB = N // RB

def _deg_body(dst_hbm, ew_hbm, zeros_hbm, out_hbm, dst_v, ew_v, acc):
    cid = lax.axis_index("c")
    sid = lax.axis_index("s")
    wid = sid * NC + cid
    row0 = sid * RPT
    pltpu.sync_copy(zeros_hbm, acc.at[pl.ds(row0, RPT)])
    plsc.subcore_barrier()

    def chunk_body(c, carry):
        pltpu.sync_copy(dst_hbm.at[wid, c], dst_v)
        pltpu.sync_copy(ew_hbm.at[wid, c], ew_v)
        pltpu.sync_copy(ew_v, acc.at[dst_v], add=True)
        return carry

    lax.fori_loop(0, CHUNKS, chunk_body, 0)
    plsc.subcore_barrier()
    pltpu.sync_copy(acc.at[pl.ds(row0, RPT)], out_hbm.at[cid, pl.ds(row0, RPT)])


def _spmm_body(hs_hbm, src_hbm, dst_hbm, ew_hbm, zeros_hbm, out_hbm,
               src_v, dst_v, ew_v, rows_v, acc, sem):
    cid = lax.axis_index("c")
    sid = lax.axis_index("s")
    wid = sid * NC + cid
    row0 = sid * RPT
    col_iotas = [lax.iota(jnp.int32, LANES) + j * LANES for j in range(D // LANES)]

    for t in range(T):
        for z in range(RPT // ZROWS):
            pltpu.sync_copy(zeros_hbm, acc.at[pl.ds(row0 + z * ZROWS, ZROWS)])
        plsc.subcore_barrier()

        def chunk_body(c, carry, t=t):
            pltpu.sync_copy(src_hbm.at[wid, c], src_v)
            pltpu.sync_copy(dst_hbm.at[wid, c], dst_v)
            pltpu.sync_copy(ew_hbm.at[wid, c], ew_v)
            pltpu.async_copy(hs_hbm.at[t].at[src_v], rows_v, sem).wait()

            def edge_body(k, inner):
                ksp = jnp.full((LANES,), k, jnp.int32)
                s = plsc.load_gather(ew_v, [ksp])
                for cols in col_iotas:
                    v = plsc.load_gather(rows_v, [ksp, cols])
                    plsc.store_scatter(rows_v, [ksp, cols], v * s)
                return inner

            lax.fori_loop(0, K, edge_body, 0)
            pltpu.sync_copy(rows_v, acc.at[dst_v], add=True)
            return carry

        lax.fori_loop(0, CHUNKS, chunk_body, 0)
        plsc.subcore_barrier()
        pltpu.sync_copy(acc.at[pl.ds(row0, RPT)],
                        out_hbm.at[cid, t, pl.ds(row0, RPT)])


@functools.lru_cache(maxsize=None)
def _sc_kernels():
    mesh = plsc.VectorSubcoreMesh(core_axis_name="c", subcore_axis_name="s",
                                  num_cores=NC, num_subcores=NS)
    params = pltpu.CompilerParams(needs_layout_passes=False)
    deg = pl.kernel(
        _deg_body,
        out_type=jax.ShapeDtypeStruct((NC, NPAD), jnp.float32),
        mesh=mesh,
        compiler_params=params,
        scratch_types=[
            pltpu.VMEM((K,), jnp.int32),
            pltpu.VMEM((K,), jnp.float32),
            pltpu.VMEM_SHARED((NPAD,), jnp.float32),
        ],
    )
    spmm = pl.kernel(
        _spmm_body,
        out_type=jax.ShapeDtypeStruct((NC, T, NPAD, D), jnp.float32),
        mesh=mesh,
        compiler_params=params,
        scratch_types=[
            pltpu.VMEM((K,), jnp.int32),
            pltpu.VMEM((K,), jnp.int32),
            pltpu.VMEM((K,), jnp.float32),
            pltpu.VMEM((K, D), jnp.float32),
            pltpu.VMEM_SHARED((NPAD, D), jnp.float32),
            pltpu.SemaphoreType.DMA,
        ],
    )
    return deg, spmm


def _tc_a_body(d0_ref, d1_ref, x_ref, w1_ref, hs_ref, dinv_ref):
    deg = d0_ref[...] + d1_ref[...] + 1.0
    dinv = jnp.where(deg > 0, lax.rsqrt(jnp.where(deg > 0, deg, 1.0)), 0.0)
    h = jnp.dot(x_ref[0], w1_ref[...], preferred_element_type=jnp.float32)
    hs_ref[0] = h * dinv
    dinv_ref[...] = dinv


def _tc_a(d0, d1, x_seq, w1):
    return pl.pallas_call(
        _tc_a_body,
        grid=(T, NB),
        in_specs=[
            pl.BlockSpec((RB, 1), lambda t, nb: (nb, 0)),
            pl.BlockSpec((RB, 1), lambda t, nb: (nb, 0)),
            pl.BlockSpec((1, RB, D), lambda t, nb: (t, nb, 0)),
            pl.BlockSpec((D, D), lambda t, nb: (0, 0)),
        ],
        out_specs=[
            pl.BlockSpec((1, RB, D), lambda t, nb: (t, nb, 0)),
            pl.BlockSpec((RB, 1), lambda t, nb: (nb, 0)),
        ],
        out_shape=[
            jax.ShapeDtypeStruct((T, N, D), jnp.float32),
            jax.ShapeDtypeStruct((N, 1), jnp.float32),
        ],
    )(d0, d1, x_seq, w1)


def _tc_b_body(s0_ref, s1_ref, hs_ref, dinv_ref, b_ref, w2_ref, hs2_ref):
    dinv = dinv_ref[...]
    x2 = jnp.maximum(dinv * (s0_ref[0, 0] + s1_ref[0, 0] + hs_ref[0]) + b_ref[...], 0.0)
    hs2_ref[0] = dinv * jnp.dot(x2, w2_ref[...], preferred_element_type=jnp.float32)


def _tc_b(s, hs1, dinv, b1, w2):
    return pl.pallas_call(
        _tc_b_body,
        grid=(T, NB),
        in_specs=[
            pl.BlockSpec((1, 1, RB, D), lambda t, nb: (0, t, nb, 0)),
            pl.BlockSpec((1, 1, RB, D), lambda t, nb: (1, t, nb, 0)),
            pl.BlockSpec((1, RB, D), lambda t, nb: (t, nb, 0)),
            pl.BlockSpec((RB, 1), lambda t, nb: (nb, 0)),
            pl.BlockSpec((1, D), lambda t, nb: (0, 0)),
            pl.BlockSpec((D, D), lambda t, nb: (0, 0)),
        ],
        out_specs=[pl.BlockSpec((1, RB, D), lambda t, nb: (t, nb, 0))],
        out_shape=[jax.ShapeDtypeStruct((T, N, D), jnp.float32)],
    )(s, s, hs1, dinv, b1, w2)[0]


def _tc_c_body(s0_ref, s1_ref, hs_ref, dinv_ref, b_ref,
               wih_ref, whh_ref, bih_ref, bhh_ref, wc_ref, bc_ref,
               out_ref, acc_ref):
    t = pl.program_id(0)
    nb = pl.program_id(1)
    dinv = dinv_ref[...]
    x3 = jnp.maximum(dinv * (s0_ref[0, 0] + s1_ref[0, 0] + hs_ref[0]) + b_ref[...], 0.0)
    colsum = jnp.sum(x3, axis=0, keepdims=True)

    @pl.when(jnp.logical_and(t == 0, nb == 0))
    def _():
        acc_ref[...] = jnp.zeros((T, D), jnp.float32)

    onehot = (lax.broadcasted_iota(jnp.int32, (T, 1), 0) == t).astype(jnp.float32)
    acc_ref[...] = acc_ref[...] + onehot * colsum

    @pl.when(jnp.logical_and(t == T - 1, nb == NB - 1))
    def _():
        seq = acc_ref[...] * (1.0 / N)
        h = jnp.zeros((1, D), jnp.float32)
        for i in range(T):
            xt = seq[i:i + 1, :]
            gi = jnp.dot(xt, wih_ref[...], preferred_element_type=jnp.float32) + bih_ref[...]
            gh = jnp.dot(h, whh_ref[...], preferred_element_type=jnp.float32) + bhh_ref[...]
            r = jax.nn.sigmoid(gi[:, :D] + gh[:, :D])
            z = jax.nn.sigmoid(gi[:, D:2 * D] + gh[:, D:2 * D])
            n = jnp.tanh(gi[:, 2 * D:] + r * gh[:, 2 * D:])
            h = (1.0 - z) * n + z * h
        out_ref[...] = jnp.dot(h, wc_ref[...], preferred_element_type=jnp.float32) + bc_ref[...]


def _tc_c(s, hs2, dinv, b2, wih_t, whh_t, bih, bhh, wc, bc):
    return pl.pallas_call(
        _tc_c_body,
        grid=(T, NB),
        in_specs=[
            pl.BlockSpec((1, 1, RB, D), lambda t, nb: (0, t, nb, 0)),
            pl.BlockSpec((1, 1, RB, D), lambda t, nb: (1, t, nb, 0)),
            pl.BlockSpec((1, RB, D), lambda t, nb: (t, nb, 0)),
            pl.BlockSpec((RB, 1), lambda t, nb: (nb, 0)),
            pl.BlockSpec((1, D), lambda t, nb: (0, 0)),
            pl.BlockSpec((D, 3 * D), lambda t, nb: (0, 0)),
            pl.BlockSpec((D, 3 * D), lambda t, nb: (0, 0)),
            pl.BlockSpec((1, 3 * D), lambda t, nb: (0, 0)),
            pl.BlockSpec((1, 3 * D), lambda t, nb: (0, 0)),
            pl.BlockSpec((D, 10), lambda t, nb: (0, 0)),
            pl.BlockSpec((1, 10), lambda t, nb: (0, 0)),
        ],
        out_specs=[pl.BlockSpec((1, 10), lambda t, nb: (0, 0))],
        out_shape=[jax.ShapeDtypeStruct((1, 10), jnp.float32)],
        scratch_shapes=[pltpu.VMEM((T, D), jnp.float32)],
    )(s, s, hs2, dinv, b2, wih_t, whh_t, bih, bhh, wc, bc)[0]


def kernel(x_seq, edge_index, edge_attr, W1, b1, W2, b2,
           W_ih, W_hh, b_ih, b_hh, Wc, bc):
    src3 = edge_index[0].reshape(NW, CHUNKS, K)
    dst3 = edge_index[1].reshape(NW, CHUNKS, K)
    ew3 = edge_attr.reshape(NW, CHUNKS, K)
    zer1 = jnp.zeros((RPT,), jnp.float32)
    zer2 = jnp.zeros((ZROWS, D), jnp.float32)

    _deg_kernel, _spmm_kernel = _sc_kernels()
    degp = _deg_kernel(dst3, ew3, zer1)
    d0 = degp[0, :N].reshape(N, 1)
    d1 = degp[1, :N].reshape(N, 1)

    hs1, dinv = _tc_a(d0, d1, x_seq, W1)
    s1 = _spmm_kernel(hs1, src3, dst3, ew3, zer2)
    hs2 = _tc_b(s1, hs1, dinv, b1.reshape(1, D), W2)
    s2 = _spmm_kernel(hs2, src3, dst3, ew3, zer2)

    out = _tc_c(s2, hs2, dinv, b2.reshape(1, D),
                W_ih.T, W_hh.T, b_ih.reshape(1, 3 * D), b_hh.reshape(1, 3 * D),
                Wc, bc.reshape(1, 10))
    return out


# speedup vs baseline: 3.7036x; 1.3797x over previous
import functools

import jax
import jax.numpy as jnp
from jax import lax
from jax.experimental import pallas as pl
from jax.experimental.pallas import tpu as pltpu
from jax.experimental.pallas import tpu_sc as plsc

T, N, E, D = 8, 10000, 320000, 128
NC, NS, LANES = 2, 16, 16
NW = NC * NS
NPAD = 10240
RPT = NPAD // NS
EPW = E // NW
K = 80
CHUNKS = EPW // K
ZROWS = 160
RB = 1000
NB = N // RB

def _deg_body(dst_hbm, ew_hbm, zeros_hbm, out_hbm, dst_v, ew_v, acc):
    cid = lax.axis_index("c")
    sid = lax.axis_index("s")
    wid = sid * NC + cid
    row0 = sid * RPT
    pltpu.sync_copy(dst_hbm.at[wid], dst_v)
    pltpu.sync_copy(ew_hbm.at[wid], ew_v)
    pltpu.sync_copy(zeros_hbm, acc.at[pl.ds(row0, RPT)])
    plsc.subcore_barrier()

    def chunk_body(c, carry):
        pltpu.sync_copy(ew_v.at[c], acc.at[dst_v.at[c]], add=True)
        return carry

    lax.fori_loop(0, CHUNKS, chunk_body, 0)
    plsc.subcore_barrier()
    pltpu.sync_copy(acc.at[pl.ds(row0, RPT)], out_hbm.at[cid, pl.ds(row0, RPT)])


def _spmm_body(hs_hbm, ed_hbm, zeros_hbm, out_hbm,
               ed0, ed1, rows0, rows1, ds0, ds1, acc,
               es0, es1, gs0, gs1, ss0, ss1):
    ed_v = [ed0, ed1]
    rows_v = [rows0, rows1]
    dst_s = [ds0, ds1]
    esem = [es0, es1]
    gsem = [gs0, gs1]
    ssem = [ss0, ss1]
    cid = lax.axis_index("c")
    sid = lax.axis_index("s")
    wid = sid * NC + cid
    row0 = sid * RPT
    col_iotas = [lax.iota(jnp.int32, LANES) + j * LANES for j in range(D // LANES)]

    def scale_rows(rv, ev):
        def edge_body(k, inner):
            ksp = jnp.full((LANES,), k, jnp.int32)
            s = plsc.bitcast(plsc.load_gather(ev, [jnp.full((LANES,), 2, jnp.int32), ksp]),
                             jnp.float32)
            for cols in col_iotas:
                v = plsc.load_gather(rv, [ksp, cols])
                plsc.store_scatter(rv, [ksp, cols], v * s)
            return inner
        lax.fori_loop(0, K, edge_body, 0)

    for t in range(T):
        for z in range(RPT // ZROWS):
            pltpu.sync_copy(zeros_hbm, acc.at[pl.ds(row0 + z * ZROWS, ZROWS)])
        plsc.subcore_barrier()

        hs_t = hs_hbm.at[t]
        pltpu.async_copy(ed_hbm.at[wid, 0], ed_v[0], esem[0])
        pltpu.async_copy(ed_hbm.at[wid, 1], ed_v[1], esem[1])
        pltpu.make_async_copy(ed_hbm.at[wid, 0], ed_v[0], esem[0]).wait()
        pltpu.async_copy(hs_t.at[ed_v[0].at[0]], rows_v[0], gsem[0])

        def chunk_body(c, carry, t=t):
            for b in range(2):
                nb = 1 - b

                @pl.when(lax.rem(c, 2) == b)
                def _():
                    pltpu.make_async_copy(hs_t.at[ed_v[b].at[0]], rows_v[b],
                                          gsem[b]).wait()
                    scale_rows(rows_v[b], ed_v[b])
                    for j in range(K // LANES):
                        sl = pl.ds(j * LANES, LANES)
                        dst_s[b][sl] = ed_v[b][1, sl]
                    pltpu.async_copy(rows_v[b], acc.at[dst_s[b]], ssem[b],
                                     add=True)

                    @pl.when(c + 2 < CHUNKS)
                    def _():
                        pltpu.async_copy(ed_hbm.at[wid, c + 2], ed_v[b], esem[b])

                    @pl.when(c + 1 < CHUNKS)
                    def _():
                        @pl.when(c > 0)
                        def _():
                            pltpu.make_async_copy(rows_v[nb], acc.at[dst_s[nb]],
                                                  ssem[nb]).wait()
                        pltpu.make_async_copy(ed_hbm.at[wid, c + 1], ed_v[nb],
                                              esem[nb]).wait()
                        pltpu.async_copy(hs_t.at[ed_v[nb].at[0]], rows_v[nb],
                                         gsem[nb])
            return carry

        lax.fori_loop(0, CHUNKS, chunk_body, 0)
        lastb = (CHUNKS - 1) % 2
        pltpu.make_async_copy(rows_v[1 - lastb], acc.at[dst_s[1 - lastb]],
                              ssem[1 - lastb]).wait()
        pltpu.make_async_copy(rows_v[lastb], acc.at[dst_s[lastb]],
                              ssem[lastb]).wait()
        plsc.subcore_barrier()
        pltpu.sync_copy(acc.at[pl.ds(row0, RPT)],
                        out_hbm.at[cid, t, pl.ds(row0, RPT)])


@functools.lru_cache(maxsize=None)
def _sc_kernels():
    mesh = plsc.VectorSubcoreMesh(core_axis_name="c", subcore_axis_name="s",
                                  num_cores=NC, num_subcores=NS)
    params = pltpu.CompilerParams(needs_layout_passes=False)
    deg = pl.kernel(
        _deg_body,
        out_type=jax.ShapeDtypeStruct((NC, NPAD), jnp.float32),
        mesh=mesh,
        compiler_params=params,
        scratch_types=[
            pltpu.VMEM((CHUNKS, K), jnp.int32),
            pltpu.VMEM((CHUNKS, K), jnp.float32),
            pltpu.VMEM_SHARED((NPAD,), jnp.float32),
        ],
    )
    spmm = pl.kernel(
        _spmm_body,
        out_type=jax.ShapeDtypeStruct((NC, T, NPAD, D), jnp.float32),
        mesh=mesh,
        compiler_params=params,
        scratch_types=[
            pltpu.VMEM((3, K), jnp.int32),
            pltpu.VMEM((3, K), jnp.int32),
            pltpu.VMEM((K, D), jnp.float32),
            pltpu.VMEM((K, D), jnp.float32),
            pltpu.VMEM((K,), jnp.int32),
            pltpu.VMEM((K,), jnp.int32),
            pltpu.VMEM_SHARED((NPAD, D), jnp.float32),
            pltpu.SemaphoreType.DMA,
            pltpu.SemaphoreType.DMA,
            pltpu.SemaphoreType.DMA,
            pltpu.SemaphoreType.DMA,
            pltpu.SemaphoreType.DMA,
            pltpu.SemaphoreType.DMA,
        ],
    )
    return deg, spmm


def _tc_a_body(d0_ref, d1_ref, x_ref, w1_ref, hs_ref, dinv_ref):
    deg = d0_ref[...] + d1_ref[...] + 1.0
    dinv = jnp.where(deg > 0, lax.rsqrt(jnp.where(deg > 0, deg, 1.0)), 0.0)
    h = jnp.dot(x_ref[0], w1_ref[...], preferred_element_type=jnp.float32)
    hs_ref[0] = h * dinv
    dinv_ref[...] = dinv


def _tc_a(d0, d1, x_seq, w1):
    return pl.pallas_call(
        _tc_a_body,
        grid=(T, NB),
        in_specs=[
            pl.BlockSpec((RB, 1), lambda t, nb: (nb, 0)),
            pl.BlockSpec((RB, 1), lambda t, nb: (nb, 0)),
            pl.BlockSpec((1, RB, D), lambda t, nb: (t, nb, 0)),
            pl.BlockSpec((D, D), lambda t, nb: (0, 0)),
        ],
        out_specs=[
            pl.BlockSpec((1, RB, D), lambda t, nb: (t, nb, 0)),
            pl.BlockSpec((RB, 1), lambda t, nb: (nb, 0)),
        ],
        out_shape=[
            jax.ShapeDtypeStruct((T, N, D), jnp.float32),
            jax.ShapeDtypeStruct((N, 1), jnp.float32),
        ],
    )(d0, d1, x_seq, w1)


def _tc_b_body(s0_ref, s1_ref, hs_ref, dinv_ref, b_ref, w2_ref, hs2_ref):
    dinv = dinv_ref[...]
    x2 = jnp.maximum(dinv * (s0_ref[0, 0] + s1_ref[0, 0] + hs_ref[0]) + b_ref[...], 0.0)
    hs2_ref[0] = dinv * jnp.dot(x2, w2_ref[...], preferred_element_type=jnp.float32)


def _tc_b(s, hs1, dinv, b1, w2):
    return pl.pallas_call(
        _tc_b_body,
        grid=(T, NB),
        in_specs=[
            pl.BlockSpec((1, 1, RB, D), lambda t, nb: (0, t, nb, 0)),
            pl.BlockSpec((1, 1, RB, D), lambda t, nb: (1, t, nb, 0)),
            pl.BlockSpec((1, RB, D), lambda t, nb: (t, nb, 0)),
            pl.BlockSpec((RB, 1), lambda t, nb: (nb, 0)),
            pl.BlockSpec((1, D), lambda t, nb: (0, 0)),
            pl.BlockSpec((D, D), lambda t, nb: (0, 0)),
        ],
        out_specs=[pl.BlockSpec((1, RB, D), lambda t, nb: (t, nb, 0))],
        out_shape=[jax.ShapeDtypeStruct((T, N, D), jnp.float32)],
    )(s, s, hs1, dinv, b1, w2)[0]


def _tc_c_body(s0_ref, s1_ref, hs_ref, dinv_ref, b_ref,
               wih_ref, whh_ref, bih_ref, bhh_ref, wc_ref, bc_ref,
               out_ref, acc_ref):
    t = pl.program_id(0)
    nb = pl.program_id(1)
    dinv = dinv_ref[...]
    x3 = jnp.maximum(dinv * (s0_ref[0, 0] + s1_ref[0, 0] + hs_ref[0]) + b_ref[...], 0.0)
    colsum = jnp.sum(x3, axis=0, keepdims=True)

    @pl.when(jnp.logical_and(t == 0, nb == 0))
    def _():
        acc_ref[...] = jnp.zeros((T, D), jnp.float32)

    onehot = (lax.broadcasted_iota(jnp.int32, (T, 1), 0) == t).astype(jnp.float32)
    acc_ref[...] = acc_ref[...] + onehot * colsum

    @pl.when(jnp.logical_and(t == T - 1, nb == NB - 1))
    def _():
        seq = acc_ref[...] * (1.0 / N)
        h = jnp.zeros((1, D), jnp.float32)
        for i in range(T):
            xt = seq[i:i + 1, :]
            gi = jnp.dot(xt, wih_ref[...], preferred_element_type=jnp.float32) + bih_ref[...]
            gh = jnp.dot(h, whh_ref[...], preferred_element_type=jnp.float32) + bhh_ref[...]
            r = jax.nn.sigmoid(gi[:, :D] + gh[:, :D])
            z = jax.nn.sigmoid(gi[:, D:2 * D] + gh[:, D:2 * D])
            n = jnp.tanh(gi[:, 2 * D:] + r * gh[:, 2 * D:])
            h = (1.0 - z) * n + z * h
        out_ref[...] = jnp.dot(h, wc_ref[...], preferred_element_type=jnp.float32) + bc_ref[...]


def _tc_c(s, hs2, dinv, b2, wih_t, whh_t, bih, bhh, wc, bc):
    return pl.pallas_call(
        _tc_c_body,
        grid=(T, NB),
        in_specs=[
            pl.BlockSpec((1, 1, RB, D), lambda t, nb: (0, t, nb, 0)),
            pl.BlockSpec((1, 1, RB, D), lambda t, nb: (1, t, nb, 0)),
            pl.BlockSpec((1, RB, D), lambda t, nb: (t, nb, 0)),
            pl.BlockSpec((RB, 1), lambda t, nb: (nb, 0)),
            pl.BlockSpec((1, D), lambda t, nb: (0, 0)),
            pl.BlockSpec((D, 3 * D), lambda t, nb: (0, 0)),
            pl.BlockSpec((D, 3 * D), lambda t, nb: (0, 0)),
            pl.BlockSpec((1, 3 * D), lambda t, nb: (0, 0)),
            pl.BlockSpec((1, 3 * D), lambda t, nb: (0, 0)),
            pl.BlockSpec((D, 10), lambda t, nb: (0, 0)),
            pl.BlockSpec((1, 10), lambda t, nb: (0, 0)),
        ],
        out_specs=[pl.BlockSpec((1, 10), lambda t, nb: (0, 0))],
        out_shape=[jax.ShapeDtypeStruct((1, 10), jnp.float32)],
        scratch_shapes=[pltpu.VMEM((T, D), jnp.float32)],
    )(s, s, hs2, dinv, b2, wih_t, whh_t, bih, bhh, wc, bc)[0]


def kernel(x_seq, edge_index, edge_attr, W1, b1, W2, b2,
           W_ih, W_hh, b_ih, b_hh, Wc, bc):
    src3 = edge_index[0].reshape(NW, CHUNKS, K)
    dst3 = edge_index[1].reshape(NW, CHUNKS, K)
    ew3 = edge_attr.reshape(NW, CHUNKS, K)
    ed4 = jnp.stack([src3, dst3, lax.bitcast_convert_type(ew3, jnp.int32)],
                    axis=2)
    zer1 = jnp.zeros((RPT,), jnp.float32)
    zer2 = jnp.zeros((ZROWS, D), jnp.float32)

    _deg_kernel, _spmm_kernel = _sc_kernels()
    degp = _deg_kernel(dst3, ew3, zer1)
    d0 = degp[0, :N].reshape(N, 1)
    d1 = degp[1, :N].reshape(N, 1)

    hs1, dinv = _tc_a(d0, d1, x_seq, W1)
    s1 = _spmm_kernel(hs1, ed4, zer2)
    hs2 = _tc_b(s1, hs1, dinv, b1.reshape(1, D), W2)
    s2 = _spmm_kernel(hs2, ed4, zer2)

    out = _tc_c(s2, hs2, dinv, b2.reshape(1, D),
                W_ih.T, W_hh.T, b_ih.reshape(1, 3 * D), b_hh.reshape(1, 3 * D),
                Wc, bc.reshape(1, 10))
    return out


# speedup vs baseline: 4.4963x; 1.2140x over previous
import functools

import jax
import jax.numpy as jnp
from jax import lax
from jax.experimental import pallas as pl
from jax.experimental.pallas import tpu as pltpu
from jax.experimental.pallas import tpu_sc as plsc

T, N, E, D = 8, 10000, 320000, 128
NC, NS, LANES = 2, 16, 16
NW = NC * NS
NPAD = 10240
RPT = NPAD // NS
EPW = E // NW
K = 80
CHUNKS = EPW // K
ZROWS = 160
RB = 1000
NB = N // RB

def _deg_body(dst_hbm, ew_hbm, zeros_hbm, out_hbm, dst_v, ew_v, acc):
    cid = lax.axis_index("c")
    sid = lax.axis_index("s")
    wid = sid * NC + cid
    row0 = sid * RPT
    pltpu.sync_copy(dst_hbm.at[wid], dst_v)
    pltpu.sync_copy(ew_hbm.at[wid], ew_v)
    pltpu.sync_copy(zeros_hbm, acc.at[pl.ds(row0, RPT)])
    plsc.subcore_barrier()

    def chunk_body(c, carry):
        pltpu.sync_copy(ew_v.at[c], acc.at[dst_v.at[c]], add=True)
        return carry

    lax.fori_loop(0, CHUNKS, chunk_body, 0)
    plsc.subcore_barrier()
    pltpu.sync_copy(acc.at[pl.ds(row0, RPT)], out_hbm.at[cid, pl.ds(row0, RPT)])


def _spmm_body(hs_hbm, ed_hbm, zeros_hbm, out_hbm,
               ed0, ed1, rows0, rows1, ds0, ds1, acc,
               es0, es1, gs0, gs1, ss0, ss1):
    ed_v = [ed0, ed1]
    rows_v = [rows0, rows1]
    dst_s = [ds0, ds1]
    esem = [es0, es1]
    gsem = [gs0, gs1]
    ssem = [ss0, ss1]
    cid = lax.axis_index("c")
    sid = lax.axis_index("s")
    wid = sid * NC + cid
    row0 = sid * RPT
    col_iotas = [lax.iota(jnp.int32, LANES) + j * LANES for j in range(D // LANES)]

    def scale_rows(rv, ev):
        def edge_body(k, inner):
            ksp = jnp.full((LANES,), k, jnp.int32)
            s = plsc.bitcast(plsc.load_gather(ev, [jnp.full((LANES,), 2, jnp.int32), ksp]),
                             jnp.float32)
            for cols in col_iotas:
                v = plsc.load_gather(rv, [ksp, cols])
                plsc.store_scatter(rv, [ksp, cols], v * s)
            return inner
        lax.fori_loop(0, K, edge_body, 0, unroll=4)

    for t in range(T):
        for z in range(RPT // ZROWS):
            pltpu.sync_copy(zeros_hbm, acc.at[pl.ds(row0 + z * ZROWS, ZROWS)])
        plsc.subcore_barrier()

        hs_t = hs_hbm.at[t]
        pltpu.async_copy(ed_hbm.at[wid, 0], ed_v[0], esem[0])
        pltpu.async_copy(ed_hbm.at[wid, 1], ed_v[1], esem[1])
        pltpu.make_async_copy(ed_hbm.at[wid, 0], ed_v[0], esem[0]).wait()
        pltpu.async_copy(hs_t.at[ed_v[0].at[0]], rows_v[0], gsem[0])

        def chunk_body(c, carry, t=t):
            for b in range(2):
                nb = 1 - b

                @pl.when(lax.rem(c, 2) == b)
                def _():
                    pltpu.make_async_copy(hs_t.at[ed_v[b].at[0]], rows_v[b],
                                          gsem[b]).wait()

                    @pl.when(c + 1 < CHUNKS)
                    def _():
                        @pl.when(c > 0)
                        def _():
                            pltpu.make_async_copy(rows_v[nb], acc.at[dst_s[nb]],
                                                  ssem[nb]).wait()
                        pltpu.make_async_copy(ed_hbm.at[wid, c + 1], ed_v[nb],
                                              esem[nb]).wait()
                        pltpu.async_copy(hs_t.at[ed_v[nb].at[0]], rows_v[nb],
                                         gsem[nb])

                    scale_rows(rows_v[b], ed_v[b])
                    for j in range(K // LANES):
                        sl = pl.ds(j * LANES, LANES)
                        dst_s[b][sl] = ed_v[b][1, sl]
                    pltpu.async_copy(rows_v[b], acc.at[dst_s[b]], ssem[b],
                                     add=True)

                    @pl.when(c + 2 < CHUNKS)
                    def _():
                        pltpu.async_copy(ed_hbm.at[wid, c + 2], ed_v[b], esem[b])
            return carry

        lax.fori_loop(0, CHUNKS, chunk_body, 0)
        lastb = (CHUNKS - 1) % 2
        pltpu.make_async_copy(rows_v[1 - lastb], acc.at[dst_s[1 - lastb]],
                              ssem[1 - lastb]).wait()
        pltpu.make_async_copy(rows_v[lastb], acc.at[dst_s[lastb]],
                              ssem[lastb]).wait()
        plsc.subcore_barrier()
        pltpu.sync_copy(acc.at[pl.ds(row0, RPT)],
                        out_hbm.at[cid, t, pl.ds(row0, RPT)])


@functools.lru_cache(maxsize=None)
def _sc_kernels():
    mesh = plsc.VectorSubcoreMesh(core_axis_name="c", subcore_axis_name="s",
                                  num_cores=NC, num_subcores=NS)
    params = pltpu.CompilerParams(needs_layout_passes=False)
    deg = pl.kernel(
        _deg_body,
        out_type=jax.ShapeDtypeStruct((NC, NPAD), jnp.float32),
        mesh=mesh,
        compiler_params=params,
        scratch_types=[
            pltpu.VMEM((CHUNKS, K), jnp.int32),
            pltpu.VMEM((CHUNKS, K), jnp.float32),
            pltpu.VMEM_SHARED((NPAD,), jnp.float32),
        ],
    )
    spmm = pl.kernel(
        _spmm_body,
        out_type=jax.ShapeDtypeStruct((NC, T, NPAD, D), jnp.float32),
        mesh=mesh,
        compiler_params=params,
        scratch_types=[
            pltpu.VMEM((3, K), jnp.int32),
            pltpu.VMEM((3, K), jnp.int32),
            pltpu.VMEM((K, D), jnp.float32),
            pltpu.VMEM((K, D), jnp.float32),
            pltpu.VMEM((K,), jnp.int32),
            pltpu.VMEM((K,), jnp.int32),
            pltpu.VMEM_SHARED((NPAD, D), jnp.float32),
            pltpu.SemaphoreType.DMA,
            pltpu.SemaphoreType.DMA,
            pltpu.SemaphoreType.DMA,
            pltpu.SemaphoreType.DMA,
            pltpu.SemaphoreType.DMA,
            pltpu.SemaphoreType.DMA,
        ],
    )
    return deg, spmm


def _tc_a_body(d0_ref, d1_ref, x_ref, w1_ref, hs_ref, dinv_ref):
    deg = d0_ref[...] + d1_ref[...] + 1.0
    dinv = jnp.where(deg > 0, lax.rsqrt(jnp.where(deg > 0, deg, 1.0)), 0.0)
    h = jnp.dot(x_ref[0], w1_ref[...], preferred_element_type=jnp.float32)
    hs_ref[0] = h * dinv
    dinv_ref[...] = dinv


def _tc_a(d0, d1, x_seq, w1):
    return pl.pallas_call(
        _tc_a_body,
        grid=(T, NB),
        in_specs=[
            pl.BlockSpec((RB, 1), lambda t, nb: (nb, 0)),
            pl.BlockSpec((RB, 1), lambda t, nb: (nb, 0)),
            pl.BlockSpec((1, RB, D), lambda t, nb: (t, nb, 0)),
            pl.BlockSpec((D, D), lambda t, nb: (0, 0)),
        ],
        out_specs=[
            pl.BlockSpec((1, RB, D), lambda t, nb: (t, nb, 0)),
            pl.BlockSpec((RB, 1), lambda t, nb: (nb, 0)),
        ],
        out_shape=[
            jax.ShapeDtypeStruct((T, N, D), jnp.float32),
            jax.ShapeDtypeStruct((N, 1), jnp.float32),
        ],
    )(d0, d1, x_seq, w1)


def _tc_b_body(s0_ref, s1_ref, hs_ref, dinv_ref, b_ref, w2_ref, hs2_ref):
    dinv = dinv_ref[...]
    x2 = jnp.maximum(dinv * (s0_ref[0, 0] + s1_ref[0, 0] + hs_ref[0]) + b_ref[...], 0.0)
    hs2_ref[0] = dinv * jnp.dot(x2, w2_ref[...], preferred_element_type=jnp.float32)


def _tc_b(s, hs1, dinv, b1, w2):
    return pl.pallas_call(
        _tc_b_body,
        grid=(T, NB),
        in_specs=[
            pl.BlockSpec((1, 1, RB, D), lambda t, nb: (0, t, nb, 0)),
            pl.BlockSpec((1, 1, RB, D), lambda t, nb: (1, t, nb, 0)),
            pl.BlockSpec((1, RB, D), lambda t, nb: (t, nb, 0)),
            pl.BlockSpec((RB, 1), lambda t, nb: (nb, 0)),
            pl.BlockSpec((1, D), lambda t, nb: (0, 0)),
            pl.BlockSpec((D, D), lambda t, nb: (0, 0)),
        ],
        out_specs=[pl.BlockSpec((1, RB, D), lambda t, nb: (t, nb, 0))],
        out_shape=[jax.ShapeDtypeStruct((T, N, D), jnp.float32)],
    )(s, s, hs1, dinv, b1, w2)[0]


def _tc_c_body(s0_ref, s1_ref, hs_ref, dinv_ref, b_ref,
               wih_ref, whh_ref, bih_ref, bhh_ref, wc_ref, bc_ref,
               out_ref, acc_ref):
    t = pl.program_id(0)
    nb = pl.program_id(1)
    dinv = dinv_ref[...]
    x3 = jnp.maximum(dinv * (s0_ref[0, 0] + s1_ref[0, 0] + hs_ref[0]) + b_ref[...], 0.0)
    colsum = jnp.sum(x3, axis=0, keepdims=True)

    @pl.when(jnp.logical_and(t == 0, nb == 0))
    def _():
        acc_ref[...] = jnp.zeros((T, D), jnp.float32)

    onehot = (lax.broadcasted_iota(jnp.int32, (T, 1), 0) == t).astype(jnp.float32)
    acc_ref[...] = acc_ref[...] + onehot * colsum

    @pl.when(jnp.logical_and(t == T - 1, nb == NB - 1))
    def _():
        seq = acc_ref[...] * (1.0 / N)
        h = jnp.zeros((1, D), jnp.float32)
        for i in range(T):
            xt = seq[i:i + 1, :]
            gi = jnp.dot(xt, wih_ref[...], preferred_element_type=jnp.float32) + bih_ref[...]
            gh = jnp.dot(h, whh_ref[...], preferred_element_type=jnp.float32) + bhh_ref[...]
            r = jax.nn.sigmoid(gi[:, :D] + gh[:, :D])
            z = jax.nn.sigmoid(gi[:, D:2 * D] + gh[:, D:2 * D])
            n = jnp.tanh(gi[:, 2 * D:] + r * gh[:, 2 * D:])
            h = (1.0 - z) * n + z * h
        out_ref[...] = jnp.dot(h, wc_ref[...], preferred_element_type=jnp.float32) + bc_ref[...]


def _tc_c(s, hs2, dinv, b2, wih_t, whh_t, bih, bhh, wc, bc):
    return pl.pallas_call(
        _tc_c_body,
        grid=(T, NB),
        in_specs=[
            pl.BlockSpec((1, 1, RB, D), lambda t, nb: (0, t, nb, 0)),
            pl.BlockSpec((1, 1, RB, D), lambda t, nb: (1, t, nb, 0)),
            pl.BlockSpec((1, RB, D), lambda t, nb: (t, nb, 0)),
            pl.BlockSpec((RB, 1), lambda t, nb: (nb, 0)),
            pl.BlockSpec((1, D), lambda t, nb: (0, 0)),
            pl.BlockSpec((D, 3 * D), lambda t, nb: (0, 0)),
            pl.BlockSpec((D, 3 * D), lambda t, nb: (0, 0)),
            pl.BlockSpec((1, 3 * D), lambda t, nb: (0, 0)),
            pl.BlockSpec((1, 3 * D), lambda t, nb: (0, 0)),
            pl.BlockSpec((D, 10), lambda t, nb: (0, 0)),
            pl.BlockSpec((1, 10), lambda t, nb: (0, 0)),
        ],
        out_specs=[pl.BlockSpec((1, 10), lambda t, nb: (0, 0))],
        out_shape=[jax.ShapeDtypeStruct((1, 10), jnp.float32)],
        scratch_shapes=[pltpu.VMEM((T, D), jnp.float32)],
    )(s, s, hs2, dinv, b2, wih_t, whh_t, bih, bhh, wc, bc)[0]


def kernel(x_seq, edge_index, edge_attr, W1, b1, W2, b2,
           W_ih, W_hh, b_ih, b_hh, Wc, bc):
    src3 = edge_index[0].reshape(NW, CHUNKS, K)
    dst3 = edge_index[1].reshape(NW, CHUNKS, K)
    ew3 = edge_attr.reshape(NW, CHUNKS, K)
    ed4 = jnp.stack([src3, dst3, lax.bitcast_convert_type(ew3, jnp.int32)],
                    axis=2)
    zer1 = jnp.zeros((RPT,), jnp.float32)
    zer2 = jnp.zeros((ZROWS, D), jnp.float32)

    _deg_kernel, _spmm_kernel = _sc_kernels()
    degp = _deg_kernel(dst3, ew3, zer1)
    d0 = degp[0, :N].reshape(N, 1)
    d1 = degp[1, :N].reshape(N, 1)

    hs1, dinv = _tc_a(d0, d1, x_seq, W1)
    s1 = _spmm_kernel(hs1, ed4, zer2)
    hs2 = _tc_b(s1, hs1, dinv, b1.reshape(1, D), W2)
    s2 = _spmm_kernel(hs2, ed4, zer2)

    out = _tc_c(s2, hs2, dinv, b2.reshape(1, D),
                W_ih.T, W_hh.T, b_ih.reshape(1, 3 * D), b_hh.reshape(1, 3 * D),
                Wc, bc.reshape(1, 10))
    return out


# speedup vs baseline: 10.8128x; 2.4048x over previous
import functools

import jax
import jax.numpy as jnp
from jax import lax
from jax.experimental import pallas as pl
from jax.experimental.pallas import tpu as pltpu
from jax.experimental.pallas import tpu_sc as plsc

T, N, E, D = 8, 10000, 320000, 128
NC, NS, LANES = 2, 16, 16
NW = NC * NS
NPAD = 10240
RPT = NPAD // NS
EPW = E // NW
K = 80
CHUNKS = EPW // K
ZROWS = 160
RB = 1000
NB = N // RB

def _deg_body(dst_hbm, ew_hbm, zeros_hbm, out_hbm, dst_v, ew_v, acc):
    cid = lax.axis_index("c")
    sid = lax.axis_index("s")
    wid = sid * NC + cid
    row0 = sid * RPT
    pltpu.sync_copy(dst_hbm.at[wid], dst_v)
    pltpu.sync_copy(ew_hbm.at[wid], ew_v)
    pltpu.sync_copy(zeros_hbm, acc.at[pl.ds(row0, RPT)])
    plsc.subcore_barrier()

    def chunk_body(c, carry):
        pltpu.sync_copy(ew_v.at[c], acc.at[dst_v.at[c]], add=True)
        return carry

    lax.fori_loop(0, CHUNKS, chunk_body, 0)
    plsc.subcore_barrier()
    pltpu.sync_copy(acc.at[pl.ds(row0, RPT)], out_hbm.at[cid, pl.ds(row0, RPT)])


def _spmm_body(hs_hbm, ed_hbm, zeros_hbm, out_hbm,
               ed0, ed1, rows0, rows1, ds0, ds1, acc,
               es0, es1, gs0, gs1, ss0, ss1):
    ed_v = [ed0, ed1]
    rows_v = [rows0, rows1]
    dst_s = [ds0, ds1]
    esem = [es0, es1]
    gsem = [gs0, gs1]
    ssem = [ss0, ss1]
    cid = lax.axis_index("c")
    sid = lax.axis_index("s")
    wid = sid * NC + cid
    row0 = sid * RPT
    col_iotas = [lax.iota(jnp.int32, LANES) + j * LANES for j in range(D // LANES)]

    def scale_rows(rv, ev):
        def edge_body(k, inner):
            ksp = jnp.full((LANES,), k, jnp.int32)
            s = plsc.bitcast(plsc.load_gather(ev, [jnp.full((LANES,), 2, jnp.int32), ksp]),
                             jnp.float32)
            for j in range(D // LANES):
                sl = pl.ds(j * LANES, LANES)
                rv[k, sl] = rv[k, sl] * s
            return inner
        lax.fori_loop(0, K, edge_body, 0, unroll=4)

    for t in range(T):
        for z in range(RPT // ZROWS):
            pltpu.sync_copy(zeros_hbm, acc.at[pl.ds(row0 + z * ZROWS, ZROWS)])
        plsc.subcore_barrier()

        hs_t = hs_hbm.at[t]
        pltpu.async_copy(ed_hbm.at[wid, 0], ed_v[0], esem[0])
        pltpu.async_copy(ed_hbm.at[wid, 1], ed_v[1], esem[1])
        pltpu.make_async_copy(ed_hbm.at[wid, 0], ed_v[0], esem[0]).wait()
        pltpu.async_copy(hs_t.at[ed_v[0].at[0]], rows_v[0], gsem[0])

        def chunk_body(c, carry, t=t):
            for b in range(2):
                nb = 1 - b

                @pl.when(lax.rem(c, 2) == b)
                def _():
                    pltpu.make_async_copy(hs_t.at[ed_v[b].at[0]], rows_v[b],
                                          gsem[b]).wait()

                    @pl.when(c + 1 < CHUNKS)
                    def _():
                        @pl.when(c > 0)
                        def _():
                            pltpu.make_async_copy(rows_v[nb], acc.at[dst_s[nb]],
                                                  ssem[nb]).wait()
                        pltpu.make_async_copy(ed_hbm.at[wid, c + 1], ed_v[nb],
                                              esem[nb]).wait()
                        pltpu.async_copy(hs_t.at[ed_v[nb].at[0]], rows_v[nb],
                                         gsem[nb])

                    scale_rows(rows_v[b], ed_v[b])
                    for j in range(K // LANES):
                        sl = pl.ds(j * LANES, LANES)
                        dst_s[b][sl] = ed_v[b][1, sl]
                    pltpu.async_copy(rows_v[b], acc.at[dst_s[b]], ssem[b],
                                     add=True)

                    @pl.when(c + 2 < CHUNKS)
                    def _():
                        pltpu.async_copy(ed_hbm.at[wid, c + 2], ed_v[b], esem[b])
            return carry

        lax.fori_loop(0, CHUNKS, chunk_body, 0)
        lastb = (CHUNKS - 1) % 2
        pltpu.make_async_copy(rows_v[1 - lastb], acc.at[dst_s[1 - lastb]],
                              ssem[1 - lastb]).wait()
        pltpu.make_async_copy(rows_v[lastb], acc.at[dst_s[lastb]],
                              ssem[lastb]).wait()
        plsc.subcore_barrier()
        pltpu.sync_copy(acc.at[pl.ds(row0, RPT)],
                        out_hbm.at[cid, t, pl.ds(row0, RPT)])


@functools.lru_cache(maxsize=None)
def _sc_kernels():
    mesh = plsc.VectorSubcoreMesh(core_axis_name="c", subcore_axis_name="s",
                                  num_cores=NC, num_subcores=NS)
    params = pltpu.CompilerParams(needs_layout_passes=False)
    deg = pl.kernel(
        _deg_body,
        out_type=jax.ShapeDtypeStruct((NC, NPAD), jnp.float32),
        mesh=mesh,
        compiler_params=params,
        scratch_types=[
            pltpu.VMEM((CHUNKS, K), jnp.int32),
            pltpu.VMEM((CHUNKS, K), jnp.float32),
            pltpu.VMEM_SHARED((NPAD,), jnp.float32),
        ],
    )
    spmm = pl.kernel(
        _spmm_body,
        out_type=jax.ShapeDtypeStruct((NC, T, NPAD, D), jnp.float32),
        mesh=mesh,
        compiler_params=params,
        scratch_types=[
            pltpu.VMEM((3, K), jnp.int32),
            pltpu.VMEM((3, K), jnp.int32),
            pltpu.VMEM((K, D), jnp.float32),
            pltpu.VMEM((K, D), jnp.float32),
            pltpu.VMEM((K,), jnp.int32),
            pltpu.VMEM((K,), jnp.int32),
            pltpu.VMEM_SHARED((NPAD, D), jnp.float32),
            pltpu.SemaphoreType.DMA,
            pltpu.SemaphoreType.DMA,
            pltpu.SemaphoreType.DMA,
            pltpu.SemaphoreType.DMA,
            pltpu.SemaphoreType.DMA,
            pltpu.SemaphoreType.DMA,
        ],
    )
    return deg, spmm


def _tc_a_body(d0_ref, d1_ref, x_ref, w1_ref, hs_ref, dinv_ref):
    deg = d0_ref[...] + d1_ref[...] + 1.0
    dinv = jnp.where(deg > 0, lax.rsqrt(jnp.where(deg > 0, deg, 1.0)), 0.0)
    h = jnp.dot(x_ref[0], w1_ref[...], preferred_element_type=jnp.float32)
    hs_ref[0] = h * dinv
    dinv_ref[...] = dinv


def _tc_a(d0, d1, x_seq, w1):
    return pl.pallas_call(
        _tc_a_body,
        grid=(T, NB),
        in_specs=[
            pl.BlockSpec((RB, 1), lambda t, nb: (nb, 0)),
            pl.BlockSpec((RB, 1), lambda t, nb: (nb, 0)),
            pl.BlockSpec((1, RB, D), lambda t, nb: (t, nb, 0)),
            pl.BlockSpec((D, D), lambda t, nb: (0, 0)),
        ],
        out_specs=[
            pl.BlockSpec((1, RB, D), lambda t, nb: (t, nb, 0)),
            pl.BlockSpec((RB, 1), lambda t, nb: (nb, 0)),
        ],
        out_shape=[
            jax.ShapeDtypeStruct((T, N, D), jnp.float32),
            jax.ShapeDtypeStruct((N, 1), jnp.float32),
        ],
    )(d0, d1, x_seq, w1)


def _tc_b_body(s0_ref, s1_ref, hs_ref, dinv_ref, b_ref, w2_ref, hs2_ref):
    dinv = dinv_ref[...]
    x2 = jnp.maximum(dinv * (s0_ref[0, 0] + s1_ref[0, 0] + hs_ref[0]) + b_ref[...], 0.0)
    hs2_ref[0] = dinv * jnp.dot(x2, w2_ref[...], preferred_element_type=jnp.float32)


def _tc_b(s, hs1, dinv, b1, w2):
    return pl.pallas_call(
        _tc_b_body,
        grid=(T, NB),
        in_specs=[
            pl.BlockSpec((1, 1, RB, D), lambda t, nb: (0, t, nb, 0)),
            pl.BlockSpec((1, 1, RB, D), lambda t, nb: (1, t, nb, 0)),
            pl.BlockSpec((1, RB, D), lambda t, nb: (t, nb, 0)),
            pl.BlockSpec((RB, 1), lambda t, nb: (nb, 0)),
            pl.BlockSpec((1, D), lambda t, nb: (0, 0)),
            pl.BlockSpec((D, D), lambda t, nb: (0, 0)),
        ],
        out_specs=[pl.BlockSpec((1, RB, D), lambda t, nb: (t, nb, 0))],
        out_shape=[jax.ShapeDtypeStruct((T, N, D), jnp.float32)],
    )(s, s, hs1, dinv, b1, w2)[0]


def _tc_c_body(s0_ref, s1_ref, hs_ref, dinv_ref, b_ref,
               wih_ref, whh_ref, bih_ref, bhh_ref, wc_ref, bc_ref,
               out_ref, acc_ref):
    t = pl.program_id(0)
    nb = pl.program_id(1)
    dinv = dinv_ref[...]
    x3 = jnp.maximum(dinv * (s0_ref[0, 0] + s1_ref[0, 0] + hs_ref[0]) + b_ref[...], 0.0)
    colsum = jnp.sum(x3, axis=0, keepdims=True)

    @pl.when(jnp.logical_and(t == 0, nb == 0))
    def _():
        acc_ref[...] = jnp.zeros((T, D), jnp.float32)

    onehot = (lax.broadcasted_iota(jnp.int32, (T, 1), 0) == t).astype(jnp.float32)
    acc_ref[...] = acc_ref[...] + onehot * colsum

    @pl.when(jnp.logical_and(t == T - 1, nb == NB - 1))
    def _():
        seq = acc_ref[...] * (1.0 / N)
        h = jnp.zeros((1, D), jnp.float32)
        for i in range(T):
            xt = seq[i:i + 1, :]
            gi = jnp.dot(xt, wih_ref[...], preferred_element_type=jnp.float32) + bih_ref[...]
            gh = jnp.dot(h, whh_ref[...], preferred_element_type=jnp.float32) + bhh_ref[...]
            r = jax.nn.sigmoid(gi[:, :D] + gh[:, :D])
            z = jax.nn.sigmoid(gi[:, D:2 * D] + gh[:, D:2 * D])
            n = jnp.tanh(gi[:, 2 * D:] + r * gh[:, 2 * D:])
            h = (1.0 - z) * n + z * h
        out_ref[...] = jnp.dot(h, wc_ref[...], preferred_element_type=jnp.float32) + bc_ref[...]


def _tc_c(s, hs2, dinv, b2, wih_t, whh_t, bih, bhh, wc, bc):
    return pl.pallas_call(
        _tc_c_body,
        grid=(T, NB),
        in_specs=[
            pl.BlockSpec((1, 1, RB, D), lambda t, nb: (0, t, nb, 0)),
            pl.BlockSpec((1, 1, RB, D), lambda t, nb: (1, t, nb, 0)),
            pl.BlockSpec((1, RB, D), lambda t, nb: (t, nb, 0)),
            pl.BlockSpec((RB, 1), lambda t, nb: (nb, 0)),
            pl.BlockSpec((1, D), lambda t, nb: (0, 0)),
            pl.BlockSpec((D, 3 * D), lambda t, nb: (0, 0)),
            pl.BlockSpec((D, 3 * D), lambda t, nb: (0, 0)),
            pl.BlockSpec((1, 3 * D), lambda t, nb: (0, 0)),
            pl.BlockSpec((1, 3 * D), lambda t, nb: (0, 0)),
            pl.BlockSpec((D, 10), lambda t, nb: (0, 0)),
            pl.BlockSpec((1, 10), lambda t, nb: (0, 0)),
        ],
        out_specs=[pl.BlockSpec((1, 10), lambda t, nb: (0, 0))],
        out_shape=[jax.ShapeDtypeStruct((1, 10), jnp.float32)],
        scratch_shapes=[pltpu.VMEM((T, D), jnp.float32)],
    )(s, s, hs2, dinv, b2, wih_t, whh_t, bih, bhh, wc, bc)[0]


def kernel(x_seq, edge_index, edge_attr, W1, b1, W2, b2,
           W_ih, W_hh, b_ih, b_hh, Wc, bc):
    src3 = edge_index[0].reshape(NW, CHUNKS, K)
    dst3 = edge_index[1].reshape(NW, CHUNKS, K)
    ew3 = edge_attr.reshape(NW, CHUNKS, K)
    ed4 = jnp.stack([src3, dst3, lax.bitcast_convert_type(ew3, jnp.int32)],
                    axis=2)
    zer1 = jnp.zeros((RPT,), jnp.float32)
    zer2 = jnp.zeros((ZROWS, D), jnp.float32)

    _deg_kernel, _spmm_kernel = _sc_kernels()
    degp = _deg_kernel(dst3, ew3, zer1)
    d0 = degp[0, :N].reshape(N, 1)
    d1 = degp[1, :N].reshape(N, 1)

    hs1, dinv = _tc_a(d0, d1, x_seq, W1)
    s1 = _spmm_kernel(hs1, ed4, zer2)
    hs2 = _tc_b(s1, hs1, dinv, b1.reshape(1, D), W2)
    s2 = _spmm_kernel(hs2, ed4, zer2)

    out = _tc_c(s2, hs2, dinv, b2.reshape(1, D),
                W_ih.T, W_hh.T, b_ih.reshape(1, 3 * D), b_hh.reshape(1, 3 * D),
                Wc, bc.reshape(1, 10))
    return out


# speedup vs baseline: 11.6260x; 1.0752x over previous
import functools

import jax
import jax.numpy as jnp
from jax import lax
from jax.experimental import pallas as pl
from jax.experimental.pallas import tpu as pltpu
from jax.experimental.pallas import tpu_sc as plsc

T, N, E, D = 8, 10000, 320000, 128
NC, NS, LANES = 2, 16, 16
NW = NC * NS
NPAD = 10240
RPT = NPAD // NS
EPW = E // NW
K = 100
CHUNKS = EPW // K
ZROWS = 160
RB = 1000
NB = N // RB

def _deg_body(dst_hbm, ew_hbm, zeros_hbm, out_hbm, dst_v, ew_v, acc):
    cid = lax.axis_index("c")
    sid = lax.axis_index("s")
    wid = sid * NC + cid
    row0 = sid * RPT
    pltpu.sync_copy(dst_hbm.at[wid], dst_v)
    pltpu.sync_copy(ew_hbm.at[wid], ew_v)
    pltpu.sync_copy(zeros_hbm, acc.at[pl.ds(row0, RPT)])
    plsc.subcore_barrier()

    def chunk_body(c, carry):
        pltpu.sync_copy(ew_v.at[c], acc.at[dst_v.at[c]], add=True)
        return carry

    lax.fori_loop(0, CHUNKS, chunk_body, 0)
    plsc.subcore_barrier()
    pltpu.sync_copy(acc.at[pl.ds(row0, RPT)], out_hbm.at[cid, pl.ds(row0, RPT)])


def _spmm_body(hs_hbm, ed_hbm, zeros_hbm, out_hbm,
               ed0, ed1, rows0, rows1, ds0, ds1, acc,
               es0, es1, gs0, gs1, ss0, ss1):
    ed_v = [ed0, ed1]
    rows_v = [rows0, rows1]
    dst_s = [ds0, ds1]
    esem = [es0, es1]
    gsem = [gs0, gs1]
    ssem = [ss0, ss1]
    cid = lax.axis_index("c")
    sid = lax.axis_index("s")
    wid = sid * NC + cid
    row0 = sid * RPT
    col_iotas = [lax.iota(jnp.int32, LANES) + j * LANES for j in range(D // LANES)]

    def scale_rows(rv, ev):
        def edge_body(k, inner):
            ksp = jnp.full((LANES,), k, jnp.int32)
            s = plsc.bitcast(plsc.load_gather(ev, [jnp.full((LANES,), 2, jnp.int32), ksp]),
                             jnp.float32)
            for j in range(D // LANES):
                sl = pl.ds(j * LANES, LANES)
                rv[k, sl] = rv[k, sl] * s
            return inner
        lax.fori_loop(0, K, edge_body, 0, unroll=4)

    for t in range(T):
        for z in range(RPT // ZROWS):
            pltpu.async_copy(zeros_hbm, acc.at[pl.ds(row0 + z * ZROWS, ZROWS)],
                             esem[z % 2])
        for z in range(RPT // ZROWS):
            pltpu.make_async_copy(zeros_hbm, acc.at[pl.ds(row0 + z * ZROWS, ZROWS)],
                                  esem[z % 2]).wait()
        plsc.subcore_barrier()

        hs_t = hs_hbm.at[t]
        pltpu.async_copy(ed_hbm.at[wid, 0], ed_v[0], esem[0])
        pltpu.async_copy(ed_hbm.at[wid, 1], ed_v[1], esem[1])
        pltpu.make_async_copy(ed_hbm.at[wid, 0], ed_v[0], esem[0]).wait()
        pltpu.async_copy(hs_t.at[ed_v[0].at[0]], rows_v[0], gsem[0])

        def chunk_body(c, carry, t=t):
            for b in range(2):
                nb = 1 - b

                @pl.when(lax.rem(c, 2) == b)
                def _():
                    pltpu.make_async_copy(hs_t.at[ed_v[b].at[0]], rows_v[b],
                                          gsem[b]).wait()

                    @pl.when(c + 1 < CHUNKS)
                    def _():
                        @pl.when(c > 0)
                        def _():
                            pltpu.make_async_copy(rows_v[nb], acc.at[dst_s[nb]],
                                                  ssem[nb]).wait()
                        pltpu.make_async_copy(ed_hbm.at[wid, c + 1], ed_v[nb],
                                              esem[nb]).wait()
                        pltpu.async_copy(hs_t.at[ed_v[nb].at[0]], rows_v[nb],
                                         gsem[nb])

                    scale_rows(rows_v[b], ed_v[b])
                    offs = [j * LANES for j in range(K // LANES)]
                    if K % LANES:
                        offs.append(K - LANES)
                    for o in offs:
                        sl = pl.ds(o, LANES)
                        dst_s[b][sl] = ed_v[b][1, sl]
                    pltpu.async_copy(rows_v[b], acc.at[dst_s[b]], ssem[b],
                                     add=True)

                    @pl.when(c + 2 < CHUNKS)
                    def _():
                        pltpu.async_copy(ed_hbm.at[wid, c + 2], ed_v[b], esem[b])
            return carry

        lax.fori_loop(0, CHUNKS, chunk_body, 0)
        lastb = (CHUNKS - 1) % 2
        pltpu.make_async_copy(rows_v[1 - lastb], acc.at[dst_s[1 - lastb]],
                              ssem[1 - lastb]).wait()
        pltpu.make_async_copy(rows_v[lastb], acc.at[dst_s[lastb]],
                              ssem[lastb]).wait()
        plsc.subcore_barrier()
        pltpu.sync_copy(acc.at[pl.ds(row0, RPT)],
                        out_hbm.at[cid, t, pl.ds(row0, RPT)])


@functools.lru_cache(maxsize=None)
def _sc_kernels():
    mesh = plsc.VectorSubcoreMesh(core_axis_name="c", subcore_axis_name="s",
                                  num_cores=NC, num_subcores=NS)
    params = pltpu.CompilerParams(needs_layout_passes=False)
    deg = pl.kernel(
        _deg_body,
        out_type=jax.ShapeDtypeStruct((NC, NPAD), jnp.float32),
        mesh=mesh,
        compiler_params=params,
        scratch_types=[
            pltpu.VMEM((CHUNKS, K), jnp.int32),
            pltpu.VMEM((CHUNKS, K), jnp.float32),
            pltpu.VMEM_SHARED((NPAD,), jnp.float32),
        ],
    )
    spmm = pl.kernel(
        _spmm_body,
        out_type=jax.ShapeDtypeStruct((NC, T, NPAD, D), jnp.float32),
        mesh=mesh,
        compiler_params=params,
        scratch_types=[
            pltpu.VMEM((4, K), jnp.int32),
            pltpu.VMEM((4, K), jnp.int32),
            pltpu.VMEM((K, D), jnp.float32),
            pltpu.VMEM((K, D), jnp.float32),
            pltpu.VMEM((K,), jnp.int32),
            pltpu.VMEM((K,), jnp.int32),
            pltpu.VMEM_SHARED((NPAD, D), jnp.float32),
            pltpu.SemaphoreType.DMA,
            pltpu.SemaphoreType.DMA,
            pltpu.SemaphoreType.DMA,
            pltpu.SemaphoreType.DMA,
            pltpu.SemaphoreType.DMA,
            pltpu.SemaphoreType.DMA,
        ],
    )
    return deg, spmm


def _tc_a_body(d0_ref, d1_ref, x_ref, w1_ref, hs_ref, dinv_ref):
    deg = d0_ref[...] + d1_ref[...] + 1.0
    dinv = jnp.where(deg > 0, lax.rsqrt(jnp.where(deg > 0, deg, 1.0)), 0.0)
    h = jnp.dot(x_ref[0], w1_ref[...], preferred_element_type=jnp.float32)
    hs_ref[0] = h * dinv
    dinv_ref[...] = dinv


def _tc_a(d0, d1, x_seq, w1):
    return pl.pallas_call(
        _tc_a_body,
        grid=(T, NB),
        in_specs=[
            pl.BlockSpec((RB, 1), lambda t, nb: (nb, 0)),
            pl.BlockSpec((RB, 1), lambda t, nb: (nb, 0)),
            pl.BlockSpec((1, RB, D), lambda t, nb: (t, nb, 0)),
            pl.BlockSpec((D, D), lambda t, nb: (0, 0)),
        ],
        out_specs=[
            pl.BlockSpec((1, RB, D), lambda t, nb: (t, nb, 0)),
            pl.BlockSpec((RB, 1), lambda t, nb: (nb, 0)),
        ],
        out_shape=[
            jax.ShapeDtypeStruct((T, N, D), jnp.float32),
            jax.ShapeDtypeStruct((N, 1), jnp.float32),
        ],
    )(d0, d1, x_seq, w1)


def _tc_b_body(s0_ref, s1_ref, hs_ref, dinv_ref, b_ref, w2_ref, hs2_ref):
    dinv = dinv_ref[...]
    x2 = jnp.maximum(dinv * (s0_ref[0, 0] + s1_ref[0, 0] + hs_ref[0]) + b_ref[...], 0.0)
    hs2_ref[0] = dinv * jnp.dot(x2, w2_ref[...], preferred_element_type=jnp.float32)


def _tc_b(s, hs1, dinv, b1, w2):
    return pl.pallas_call(
        _tc_b_body,
        grid=(T, NB),
        in_specs=[
            pl.BlockSpec((1, 1, RB, D), lambda t, nb: (0, t, nb, 0)),
            pl.BlockSpec((1, 1, RB, D), lambda t, nb: (1, t, nb, 0)),
            pl.BlockSpec((1, RB, D), lambda t, nb: (t, nb, 0)),
            pl.BlockSpec((RB, 1), lambda t, nb: (nb, 0)),
            pl.BlockSpec((1, D), lambda t, nb: (0, 0)),
            pl.BlockSpec((D, D), lambda t, nb: (0, 0)),
        ],
        out_specs=[pl.BlockSpec((1, RB, D), lambda t, nb: (t, nb, 0))],
        out_shape=[jax.ShapeDtypeStruct((T, N, D), jnp.float32)],
    )(s, s, hs1, dinv, b1, w2)[0]


def _tc_c_body(s0_ref, s1_ref, hs_ref, dinv_ref, b_ref,
               wih_ref, whh_ref, bih_ref, bhh_ref, wc_ref, bc_ref,
               out_ref, acc_ref):
    t = pl.program_id(0)
    nb = pl.program_id(1)
    dinv = dinv_ref[...]
    x3 = jnp.maximum(dinv * (s0_ref[0, 0] + s1_ref[0, 0] + hs_ref[0]) + b_ref[...], 0.0)
    colsum = jnp.sum(x3, axis=0, keepdims=True)

    @pl.when(jnp.logical_and(t == 0, nb == 0))
    def _():
        acc_ref[...] = jnp.zeros((T, D), jnp.float32)

    onehot = (lax.broadcasted_iota(jnp.int32, (T, 1), 0) == t).astype(jnp.float32)
    acc_ref[...] = acc_ref[...] + onehot * colsum

    @pl.when(jnp.logical_and(t == T - 1, nb == NB - 1))
    def _():
        seq = acc_ref[...] * (1.0 / N)
        h = jnp.zeros((1, D), jnp.float32)
        for i in range(T):
            xt = seq[i:i + 1, :]
            gi = jnp.dot(xt, wih_ref[...], preferred_element_type=jnp.float32) + bih_ref[...]
            gh = jnp.dot(h, whh_ref[...], preferred_element_type=jnp.float32) + bhh_ref[...]
            r = jax.nn.sigmoid(gi[:, :D] + gh[:, :D])
            z = jax.nn.sigmoid(gi[:, D:2 * D] + gh[:, D:2 * D])
            n = jnp.tanh(gi[:, 2 * D:] + r * gh[:, 2 * D:])
            h = (1.0 - z) * n + z * h
        out_ref[...] = jnp.dot(h, wc_ref[...], preferred_element_type=jnp.float32) + bc_ref[...]


def _tc_c(s, hs2, dinv, b2, wih_t, whh_t, bih, bhh, wc, bc):
    return pl.pallas_call(
        _tc_c_body,
        grid=(T, NB),
        in_specs=[
            pl.BlockSpec((1, 1, RB, D), lambda t, nb: (0, t, nb, 0)),
            pl.BlockSpec((1, 1, RB, D), lambda t, nb: (1, t, nb, 0)),
            pl.BlockSpec((1, RB, D), lambda t, nb: (t, nb, 0)),
            pl.BlockSpec((RB, 1), lambda t, nb: (nb, 0)),
            pl.BlockSpec((1, D), lambda t, nb: (0, 0)),
            pl.BlockSpec((D, 3 * D), lambda t, nb: (0, 0)),
            pl.BlockSpec((D, 3 * D), lambda t, nb: (0, 0)),
            pl.BlockSpec((1, 3 * D), lambda t, nb: (0, 0)),
            pl.BlockSpec((1, 3 * D), lambda t, nb: (0, 0)),
            pl.BlockSpec((D, 10), lambda t, nb: (0, 0)),
            pl.BlockSpec((1, 10), lambda t, nb: (0, 0)),
        ],
        out_specs=[pl.BlockSpec((1, 10), lambda t, nb: (0, 0))],
        out_shape=[jax.ShapeDtypeStruct((1, 10), jnp.float32)],
        scratch_shapes=[pltpu.VMEM((T, D), jnp.float32)],
    )(s, s, hs2, dinv, b2, wih_t, whh_t, bih, bhh, wc, bc)[0]


def kernel(x_seq, edge_index, edge_attr, W1, b1, W2, b2,
           W_ih, W_hh, b_ih, b_hh, Wc, bc):
    src3 = edge_index[0].reshape(NW, CHUNKS, K)
    dst3 = edge_index[1].reshape(NW, CHUNKS, K)
    ew3 = edge_attr.reshape(NW, CHUNKS, K)
    ed4 = jnp.stack([src3, dst3, lax.bitcast_convert_type(ew3, jnp.int32),
                     src3], axis=2)
    zer1 = jnp.zeros((RPT,), jnp.float32)
    zer2 = jnp.zeros((ZROWS, D), jnp.float32)

    _deg_kernel, _spmm_kernel = _sc_kernels()
    degp = _deg_kernel(dst3, ew3, zer1)
    d0 = degp[0, :N].reshape(N, 1)
    d1 = degp[1, :N].reshape(N, 1)

    hs1, dinv = _tc_a(d0, d1, x_seq, W1)
    s1 = _spmm_kernel(hs1, ed4, zer2)
    hs2 = _tc_b(s1, hs1, dinv, b1.reshape(1, D), W2)
    s2 = _spmm_kernel(hs2, ed4, zer2)

    out = _tc_c(s2, hs2, dinv, b2.reshape(1, D),
                W_ih.T, W_hh.T, b_ih.reshape(1, 3 * D), b_hh.reshape(1, 3 * D),
                Wc, bc.reshape(1, 10))
    return out


# speedup vs baseline: 11.9151x; 1.0249x over previous
import functools

import jax
import jax.numpy as jnp
from jax import lax
from jax.experimental import pallas as pl
from jax.experimental.pallas import tpu as pltpu
from jax.experimental.pallas import tpu_sc as plsc

T, N, E, D = 8, 10000, 320000, 128
NC, NS, LANES = 2, 16, 16
NW = NC * NS
NPAD = 10240
RPT = NPAD // NS
EPW = E // NW
K = 100
CHUNKS = EPW // K
ZROWS = 160
RB = 2000
NB = N // RB

def _deg_body(dst_hbm, ew_hbm, zeros_hbm, out_hbm, dst_v, ew_v, acc):
    cid = lax.axis_index("c")
    sid = lax.axis_index("s")
    wid = sid * NC + cid
    row0 = sid * RPT
    pltpu.sync_copy(dst_hbm.at[wid], dst_v)
    pltpu.sync_copy(ew_hbm.at[wid], ew_v)
    pltpu.sync_copy(zeros_hbm, acc.at[pl.ds(row0, RPT)])
    plsc.subcore_barrier()

    def chunk_body(c, carry):
        pltpu.sync_copy(ew_v.at[c], acc.at[dst_v.at[c]], add=True)
        return carry

    lax.fori_loop(0, CHUNKS, chunk_body, 0)
    plsc.subcore_barrier()
    pltpu.sync_copy(acc.at[pl.ds(row0, RPT)], out_hbm.at[cid, pl.ds(row0, RPT)])


def _spmm_body(hs_hbm, ed_hbm, zeros_hbm, out_hbm,
               ed0, ed1, rows0, rows1, ds0, ds1, acc,
               es0, es1, gs0, gs1, ss0, ss1):
    ed_v = [ed0, ed1]
    rows_v = [rows0, rows1]
    dst_s = [ds0, ds1]
    esem = [es0, es1]
    gsem = [gs0, gs1]
    ssem = [ss0, ss1]
    cid = lax.axis_index("c")
    sid = lax.axis_index("s")
    wid = sid * NC + cid
    row0 = sid * RPT
    col_iotas = [lax.iota(jnp.int32, LANES) + j * LANES for j in range(D // LANES)]

    def scale_rows(rv, ev):
        def edge_body(k, inner):
            ksp = jnp.full((LANES,), k, jnp.int32)
            s = plsc.bitcast(plsc.load_gather(ev, [jnp.full((LANES,), 2, jnp.int32), ksp]),
                             jnp.float32)
            for j in range(D // LANES):
                sl = pl.ds(j * LANES, LANES)
                rv[k, sl] = rv[k, sl] * s
            return inner
        lax.fori_loop(0, K, edge_body, 0, unroll=8)

    for t in range(T):
        for z in range(RPT // ZROWS):
            pltpu.async_copy(zeros_hbm, acc.at[pl.ds(row0 + z * ZROWS, ZROWS)],
                             esem[z % 2])
        for z in range(RPT // ZROWS):
            pltpu.make_async_copy(zeros_hbm, acc.at[pl.ds(row0 + z * ZROWS, ZROWS)],
                                  esem[z % 2]).wait()
        plsc.subcore_barrier()

        hs_t = hs_hbm.at[t]
        pltpu.async_copy(ed_hbm.at[wid, 0], ed_v[0], esem[0])
        pltpu.async_copy(ed_hbm.at[wid, 1], ed_v[1], esem[1])
        pltpu.make_async_copy(ed_hbm.at[wid, 0], ed_v[0], esem[0]).wait()
        pltpu.async_copy(hs_t.at[ed_v[0].at[0]], rows_v[0], gsem[0])

        def chunk_body(c, carry, t=t):
            for b in range(2):
                nb = 1 - b

                @pl.when(lax.rem(c, 2) == b)
                def _():
                    pltpu.make_async_copy(hs_t.at[ed_v[b].at[0]], rows_v[b],
                                          gsem[b]).wait()

                    @pl.when(c + 1 < CHUNKS)
                    def _():
                        @pl.when(c > 0)
                        def _():
                            pltpu.make_async_copy(rows_v[nb], acc.at[dst_s[nb]],
                                                  ssem[nb]).wait()
                        pltpu.make_async_copy(ed_hbm.at[wid, c + 1], ed_v[nb],
                                              esem[nb]).wait()
                        pltpu.async_copy(hs_t.at[ed_v[nb].at[0]], rows_v[nb],
                                         gsem[nb])

                    scale_rows(rows_v[b], ed_v[b])
                    offs = [j * LANES for j in range(K // LANES)]
                    if K % LANES:
                        offs.append(K - LANES)
                    for o in offs:
                        sl = pl.ds(o, LANES)
                        dst_s[b][sl] = ed_v[b][1, sl]
                    pltpu.async_copy(rows_v[b], acc.at[dst_s[b]], ssem[b],
                                     add=True)

                    @pl.when(c + 2 < CHUNKS)
                    def _():
                        pltpu.async_copy(ed_hbm.at[wid, c + 2], ed_v[b], esem[b])
            return carry

        lax.fori_loop(0, CHUNKS, chunk_body, 0)
        lastb = (CHUNKS - 1) % 2
        pltpu.make_async_copy(rows_v[1 - lastb], acc.at[dst_s[1 - lastb]],
                              ssem[1 - lastb]).wait()
        pltpu.make_async_copy(rows_v[lastb], acc.at[dst_s[lastb]],
                              ssem[lastb]).wait()
        plsc.subcore_barrier()
        pltpu.sync_copy(acc.at[pl.ds(row0, RPT)],
                        out_hbm.at[cid, t, pl.ds(row0, RPT)])


@functools.lru_cache(maxsize=None)
def _sc_kernels():
    mesh = plsc.VectorSubcoreMesh(core_axis_name="c", subcore_axis_name="s",
                                  num_cores=NC, num_subcores=NS)
    params = pltpu.CompilerParams(needs_layout_passes=False)
    deg = pl.kernel(
        _deg_body,
        out_type=jax.ShapeDtypeStruct((NC, NPAD), jnp.float32),
        mesh=mesh,
        compiler_params=params,
        scratch_types=[
            pltpu.VMEM((CHUNKS, K), jnp.int32),
            pltpu.VMEM((CHUNKS, K), jnp.float32),
            pltpu.VMEM_SHARED((NPAD,), jnp.float32),
        ],
    )
    spmm = pl.kernel(
        _spmm_body,
        out_type=jax.ShapeDtypeStruct((NC, T, NPAD, D), jnp.float32),
        mesh=mesh,
        compiler_params=params,
        scratch_types=[
            pltpu.VMEM((4, K), jnp.int32),
            pltpu.VMEM((4, K), jnp.int32),
            pltpu.VMEM((K, D), jnp.float32),
            pltpu.VMEM((K, D), jnp.float32),
            pltpu.VMEM((K,), jnp.int32),
            pltpu.VMEM((K,), jnp.int32),
            pltpu.VMEM_SHARED((NPAD, D), jnp.float32),
            pltpu.SemaphoreType.DMA,
            pltpu.SemaphoreType.DMA,
            pltpu.SemaphoreType.DMA,
            pltpu.SemaphoreType.DMA,
            pltpu.SemaphoreType.DMA,
            pltpu.SemaphoreType.DMA,
        ],
    )
    return deg, spmm


def _tc_a_body(d0_ref, d1_ref, x_ref, w1_ref, hs_ref, dinv_ref):
    deg = d0_ref[...] + d1_ref[...] + 1.0
    dinv = jnp.where(deg > 0, lax.rsqrt(jnp.where(deg > 0, deg, 1.0)), 0.0)
    h = jnp.dot(x_ref[0], w1_ref[...], preferred_element_type=jnp.float32)
    hs_ref[0] = h * dinv
    dinv_ref[...] = dinv


def _tc_a(d0, d1, x_seq, w1):
    return pl.pallas_call(
        _tc_a_body,
        grid=(T, NB),
        in_specs=[
            pl.BlockSpec((RB, 1), lambda t, nb: (nb, 0)),
            pl.BlockSpec((RB, 1), lambda t, nb: (nb, 0)),
            pl.BlockSpec((1, RB, D), lambda t, nb: (t, nb, 0)),
            pl.BlockSpec((D, D), lambda t, nb: (0, 0)),
        ],
        out_specs=[
            pl.BlockSpec((1, RB, D), lambda t, nb: (t, nb, 0)),
            pl.BlockSpec((RB, 1), lambda t, nb: (nb, 0)),
        ],
        out_shape=[
            jax.ShapeDtypeStruct((T, N, D), jnp.float32),
            jax.ShapeDtypeStruct((N, 1), jnp.float32),
        ],
    )(d0, d1, x_seq, w1)


def _tc_b_body(s0_ref, s1_ref, hs_ref, dinv_ref, b_ref, w2_ref, hs2_ref):
    dinv = dinv_ref[...]
    x2 = jnp.maximum(dinv * (s0_ref[0, 0] + s1_ref[0, 0] + hs_ref[0]) + b_ref[...], 0.0)
    hs2_ref[0] = dinv * jnp.dot(x2, w2_ref[...], preferred_element_type=jnp.float32)


def _tc_b(s, hs1, dinv, b1, w2):
    return pl.pallas_call(
        _tc_b_body,
        grid=(T, NB),
        in_specs=[
            pl.BlockSpec((1, 1, RB, D), lambda t, nb: (0, t, nb, 0)),
            pl.BlockSpec((1, 1, RB, D), lambda t, nb: (1, t, nb, 0)),
            pl.BlockSpec((1, RB, D), lambda t, nb: (t, nb, 0)),
            pl.BlockSpec((RB, 1), lambda t, nb: (nb, 0)),
            pl.BlockSpec((1, D), lambda t, nb: (0, 0)),
            pl.BlockSpec((D, D), lambda t, nb: (0, 0)),
        ],
        out_specs=[pl.BlockSpec((1, RB, D), lambda t, nb: (t, nb, 0))],
        out_shape=[jax.ShapeDtypeStruct((T, N, D), jnp.float32)],
    )(s, s, hs1, dinv, b1, w2)[0]


def _tc_c_body(s0_ref, s1_ref, hs_ref, dinv_ref, b_ref,
               wih_ref, whh_ref, bih_ref, bhh_ref, wc_ref, bc_ref,
               out_ref, acc_ref):
    t = pl.program_id(0)
    nb = pl.program_id(1)
    dinv = dinv_ref[...]
    x3 = jnp.maximum(dinv * (s0_ref[0, 0] + s1_ref[0, 0] + hs_ref[0]) + b_ref[...], 0.0)
    colsum = jnp.sum(x3, axis=0, keepdims=True)

    @pl.when(jnp.logical_and(t == 0, nb == 0))
    def _():
        acc_ref[...] = jnp.zeros((T, D), jnp.float32)

    onehot = (lax.broadcasted_iota(jnp.int32, (T, 1), 0) == t).astype(jnp.float32)
    acc_ref[...] = acc_ref[...] + onehot * colsum

    @pl.when(jnp.logical_and(t == T - 1, nb == NB - 1))
    def _():
        seq = acc_ref[...] * (1.0 / N)
        h = jnp.zeros((1, D), jnp.float32)
        for i in range(T):
            xt = seq[i:i + 1, :]
            gi = jnp.dot(xt, wih_ref[...], preferred_element_type=jnp.float32) + bih_ref[...]
            gh = jnp.dot(h, whh_ref[...], preferred_element_type=jnp.float32) + bhh_ref[...]
            r = jax.nn.sigmoid(gi[:, :D] + gh[:, :D])
            z = jax.nn.sigmoid(gi[:, D:2 * D] + gh[:, D:2 * D])
            n = jnp.tanh(gi[:, 2 * D:] + r * gh[:, 2 * D:])
            h = (1.0 - z) * n + z * h
        out_ref[...] = jnp.dot(h, wc_ref[...], preferred_element_type=jnp.float32) + bc_ref[...]


def _tc_c(s, hs2, dinv, b2, wih_t, whh_t, bih, bhh, wc, bc):
    return pl.pallas_call(
        _tc_c_body,
        grid=(T, NB),
        in_specs=[
            pl.BlockSpec((1, 1, RB, D), lambda t, nb: (0, t, nb, 0)),
            pl.BlockSpec((1, 1, RB, D), lambda t, nb: (1, t, nb, 0)),
            pl.BlockSpec((1, RB, D), lambda t, nb: (t, nb, 0)),
            pl.BlockSpec((RB, 1), lambda t, nb: (nb, 0)),
            pl.BlockSpec((1, D), lambda t, nb: (0, 0)),
            pl.BlockSpec((D, 3 * D), lambda t, nb: (0, 0)),
            pl.BlockSpec((D, 3 * D), lambda t, nb: (0, 0)),
            pl.BlockSpec((1, 3 * D), lambda t, nb: (0, 0)),
            pl.BlockSpec((1, 3 * D), lambda t, nb: (0, 0)),
            pl.BlockSpec((D, 10), lambda t, nb: (0, 0)),
            pl.BlockSpec((1, 10), lambda t, nb: (0, 0)),
        ],
        out_specs=[pl.BlockSpec((1, 10), lambda t, nb: (0, 0))],
        out_shape=[jax.ShapeDtypeStruct((1, 10), jnp.float32)],
        scratch_shapes=[pltpu.VMEM((T, D), jnp.float32)],
    )(s, s, hs2, dinv, b2, wih_t, whh_t, bih, bhh, wc, bc)[0]


def kernel(x_seq, edge_index, edge_attr, W1, b1, W2, b2,
           W_ih, W_hh, b_ih, b_hh, Wc, bc):
    src3 = edge_index[0].reshape(NW, CHUNKS, K)
    dst3 = edge_index[1].reshape(NW, CHUNKS, K)
    ew3 = edge_attr.reshape(NW, CHUNKS, K)
    ed4 = jnp.stack([src3, dst3, lax.bitcast_convert_type(ew3, jnp.int32),
                     src3], axis=2)
    zer1 = jnp.zeros((RPT,), jnp.float32)
    zer2 = jnp.zeros((ZROWS, D), jnp.float32)

    _deg_kernel, _spmm_kernel = _sc_kernels()
    degp = _deg_kernel(dst3, ew3, zer1)
    d0 = degp[0, :N].reshape(N, 1)
    d1 = degp[1, :N].reshape(N, 1)

    hs1, dinv = _tc_a(d0, d1, x_seq, W1)
    s1 = _spmm_kernel(hs1, ed4, zer2)
    hs2 = _tc_b(s1, hs1, dinv, b1.reshape(1, D), W2)
    s2 = _spmm_kernel(hs2, ed4, zer2)

    out = _tc_c(s2, hs2, dinv, b2.reshape(1, D),
                W_ih.T, W_hh.T, b_ih.reshape(1, 3 * D), b_hh.reshape(1, 3 * D),
                Wc, bc.reshape(1, 10))
    return out
